# Initial kernel scaffold; baseline (speedup 1.0000x reference)
import jax
import jax.numpy as jnp
from jax.experimental import pallas as pl


def kernel(x, edge_index, W1, b1, W2, b2, w):
    raise NotImplementedError("write your pallas kernel here")


# speedup vs baseline: 72.7923x; 72.7923x over previous
import functools

import jax
import jax.numpy as jnp
from jax import lax
from jax.experimental import pallas as pl
from jax.experimental.pallas import tpu as pltpu
from jax.experimental.pallas import tpu_sc as plsc

NUM_CORES = 2
NUM_SUBCORES = 16
NUM_TILES = NUM_CORES * NUM_SUBCORES
LANES = 16

NUM_DIMS = 5
NUM_GROUPS = 6
CHUNK = 11200


def _sc_edge_dot_body(xcols, srcp, dstp, out, col, sbuf, dbuf, accbuf,
                      *, np_rows, ch, n_chunks):
    wid = lax.axis_index("s") * NUM_CORES + lax.axis_index("c")
    accbuf[...] = jnp.zeros((LANES,), jnp.float32)

    @pl.when(wid < NUM_DIMS * NUM_GROUPS)
    def _():
        d = wid % NUM_DIMS
        g = wid // NUM_DIMS
        pltpu.sync_copy(xcols.at[d], col)

        @pl.loop(0, n_chunks, init_carry=jnp.zeros((LANES,), jnp.float32))
        def chunk_loop(c, acc):
            base = g * ch + c * CHUNK
            pltpu.sync_copy(srcp.at[pl.ds(base, CHUNK)], sbuf)
            pltpu.sync_copy(dstp.at[pl.ds(base, CHUNK)], dbuf)

            @pl.loop(0, CHUNK // LANES, init_carry=acc)
            def vec_loop(i, a):
                off = i * LANES
                sv = sbuf[pl.ds(off, LANES)]
                tv = dbuf[pl.ds(off, LANES)]
                return a + (plsc.load_gather(col, [sv])
                            * plsc.load_gather(col, [tv]))

            return vec_loop

        accbuf[...] = chunk_loop

    pltpu.sync_copy(accbuf, out.at[wid])


def _finish_body(p_ref, w_ref, o_ref):
    o_ref[0] = jnp.sum(p_ref[...]) * w_ref[0]


def kernel(x, edge_index, W1, b1, W2, b2, w):
    n = x.shape[0]
    e = edge_index.shape[1]
    np_rows = ((n + 1 + LANES - 1) // LANES) * LANES
    ch = -(-e // (NUM_GROUPS * CHUNK)) * CHUNK
    n_chunks = ch // CHUNK
    ep = NUM_GROUPS * ch

    xcols = jnp.pad(x.astype(jnp.float32).T, ((0, 0), (0, np_rows - n)))
    pad = jnp.full((ep - e,), n, dtype=jnp.int32)
    srcp = jnp.concatenate([edge_index[0].astype(jnp.int32), pad])
    dstp = jnp.concatenate([edge_index[1].astype(jnp.int32), pad])

    sc_call = pl.kernel(
        functools.partial(_sc_edge_dot_body, np_rows=np_rows, ch=ch,
                          n_chunks=n_chunks),
        out_type=jax.ShapeDtypeStruct((NUM_TILES, LANES), jnp.float32),
        mesh=plsc.VectorSubcoreMesh(core_axis_name="c", subcore_axis_name="s"),
        compiler_params=pltpu.CompilerParams(needs_layout_passes=False),
        scratch_types=[
            pltpu.VMEM((np_rows,), jnp.float32),
            pltpu.VMEM((CHUNK,), jnp.int32),
            pltpu.VMEM((CHUNK,), jnp.int32),
            pltpu.VMEM((LANES,), jnp.float32),
        ],
    )
    partials = sc_call(xcols, srcp, dstp)

    finish = pl.pallas_call(
        _finish_body,
        out_shape=jax.ShapeDtypeStruct((1,), jnp.float32),
        in_specs=[
            pl.BlockSpec(memory_space=pltpu.VMEM),
            pl.BlockSpec(memory_space=pltpu.SMEM),
        ],
        out_specs=pl.BlockSpec(memory_space=pltpu.SMEM),
    )
    return finish(partials, w.astype(jnp.float32))


# speedup vs baseline: 150.4181x; 2.0664x over previous
import functools

import jax
import jax.numpy as jnp
from jax import lax
from jax.experimental import pallas as pl
from jax.experimental.pallas import tpu as pltpu
from jax.experimental.pallas import tpu_sc as plsc

NUM_CORES = 2
NUM_SUBCORES = 16
NUM_TILES = NUM_CORES * NUM_SUBCORES
LANES = 16

NUM_DIMS = 5
NUM_GROUPS = 6
CHUNK = 16800
UNROLL = 8


def _sc_edge_dot_body(xcols, srcp, dstp, out, col, sbuf0, sbuf1, dbuf0,
                      dbuf1, accbuf, ssem, dsem, *, ch, n_chunks):
    sbufs = (sbuf0, sbuf1)
    dbufs = (dbuf0, dbuf1)
    wid = lax.axis_index("s") * NUM_CORES + lax.axis_index("c")
    accbuf[...] = jnp.zeros((LANES,), jnp.float32)

    @pl.when(wid < NUM_DIMS * NUM_GROUPS)
    def _():
        d = wid % NUM_DIMS
        g = wid // NUM_DIMS
        pltpu.sync_copy(xcols.at[d], col)

        def edge_copies(c, slot):
            base = g * ch + c * CHUNK
            return (
                pltpu.make_async_copy(srcp.at[pl.ds(base, CHUNK)],
                                      sbufs[slot], ssem.at[slot]),
                pltpu.make_async_copy(dstp.at[pl.ds(base, CHUNK)],
                                      dbufs[slot], dsem.at[slot]),
            )

        def start(c, slot):
            for cp in edge_copies(c, slot):
                cp.start()

        def wait(c, slot):
            for cp in edge_copies(c, slot):
                cp.wait()

        def compute(slot, acc):
            sb = sbufs[slot]
            db = dbufs[slot]

            @plsc.parallel_loop(0, CHUNK, step=LANES, unroll=UNROLL,
                                carry=acc)
            def vec_loop(off, a):
                sv = sb[pl.ds(off, LANES)]
                tv = db[pl.ds(off, LANES)]
                return a + (plsc.load_gather(col, [sv])
                            * plsc.load_gather(col, [tv]))

            return vec_loop

        start(0, 0)

        @pl.loop(0, n_chunks // 2, init_carry=jnp.zeros((LANES,), jnp.float32))
        def chunk_loop(cc, acc):
            c0 = cc * 2
            start(c0 + 1, 1)
            wait(c0, 0)
            acc = compute(0, acc)

            @pl.when(c0 + 2 < n_chunks)
            def _():
                start(c0 + 2, 0)

            wait(c0 + 1, 1)
            return compute(1, acc)

        accbuf[...] = chunk_loop

    pltpu.sync_copy(accbuf, out.at[wid])


def _finish_body(p_ref, w_ref, o_ref):
    o_ref[0] = jnp.sum(p_ref[...]) * w_ref[0]


def kernel(x, edge_index, W1, b1, W2, b2, w):
    n = x.shape[0]
    e = edge_index.shape[1]
    np_rows = ((n + 1 + LANES - 1) // LANES) * LANES
    ch = -(-e // (NUM_GROUPS * 2 * CHUNK)) * 2 * CHUNK
    n_chunks = ch // CHUNK
    ep = NUM_GROUPS * ch

    xcols = jnp.pad(x.astype(jnp.float32).T, ((0, 0), (0, np_rows - n)))
    pad = jnp.full((ep - e,), n, dtype=jnp.int32)
    srcp = jnp.concatenate([edge_index[0].astype(jnp.int32), pad])
    dstp = jnp.concatenate([edge_index[1].astype(jnp.int32), pad])

    sc_call = pl.kernel(
        functools.partial(_sc_edge_dot_body, ch=ch, n_chunks=n_chunks),
        out_type=jax.ShapeDtypeStruct((NUM_TILES, LANES), jnp.float32),
        mesh=plsc.VectorSubcoreMesh(core_axis_name="c", subcore_axis_name="s"),
        compiler_params=pltpu.CompilerParams(needs_layout_passes=False),
        scratch_types=[
            pltpu.VMEM((np_rows,), jnp.float32),
            pltpu.VMEM((CHUNK,), jnp.int32),
            pltpu.VMEM((CHUNK,), jnp.int32),
            pltpu.VMEM((CHUNK,), jnp.int32),
            pltpu.VMEM((CHUNK,), jnp.int32),
            pltpu.VMEM((LANES,), jnp.float32),
            pltpu.SemaphoreType.DMA((2,)),
            pltpu.SemaphoreType.DMA((2,)),
        ],
    )
    partials = sc_call(xcols, srcp, dstp)

    finish = pl.pallas_call(
        _finish_body,
        out_shape=jax.ShapeDtypeStruct((1,), jnp.float32),
        in_specs=[
            pl.BlockSpec(memory_space=pltpu.VMEM),
            pl.BlockSpec(memory_space=pltpu.SMEM),
        ],
        out_specs=pl.BlockSpec(memory_space=pltpu.SMEM),
    )
    return finish(partials, w.astype(jnp.float32))


# speedup vs baseline: 185.7180x; 1.2347x over previous
import functools

import jax
import jax.numpy as jnp
from jax import lax
from jax.experimental import pallas as pl
from jax.experimental.pallas import tpu as pltpu
from jax.experimental.pallas import tpu_sc as plsc

NUM_CORES = 2
NUM_SUBCORES = 16
NUM_TILES = NUM_CORES * NUM_SUBCORES
LANES = 16

NUM_DIMS = 5
PAIR_TILES = 12
SINGLE_TILES = 8
CHUNK = 7680
UNROLL = 8


def _sc_edge_dot_body(xcols, ei, out, col0, col1, sbuf0, sbuf1, dbuf0,
                      dbuf1, accbuf, ssem, dsem, *, n_chunks, tail, e, np_rows):
    sbufs = (sbuf0, sbuf1)
    dbufs = (dbuf0, dbuf1)
    wid = lax.axis_index("s") * NUM_CORES + lax.axis_index("c")
    accbuf[...] = jnp.zeros((LANES,), jnp.float32)

    def edge_copies(base, slot, size):
        return (
            pltpu.make_async_copy(ei.at[pl.ds(base, size)],
                                  sbufs[slot].at[pl.ds(0, size)],
                                  ssem.at[slot]),
            pltpu.make_async_copy(ei.at[pl.ds(e + base, size)],
                                  dbufs[slot].at[pl.ds(0, size)],
                                  dsem.at[slot]),
        )

    def start(base, slot, size=CHUNK):
        for cp in edge_copies(base, slot, size):
            cp.start()

    def wait(base, slot, size=CHUNK):
        for cp in edge_copies(base, slot, size):
            cp.wait()

    def compute(slot, acc, cols, mask, n_edges=CHUNK):
        sb = sbufs[slot]
        db = dbufs[slot]

        @plsc.parallel_loop(0, n_edges, step=LANES, unroll=UNROLL, carry=acc)
        def vec_loop(off, a):
            sv = sb[pl.ds(off, LANES)]
            tv = db[pl.ds(off, LANES)]
            for col in cols:
                a = a + (plsc.load_gather(col, [sv], mask=mask)
                         * plsc.load_gather(col, [tv], mask=mask))
            return a

        return vec_loop

    def run_class(rank, stride, cols):
        ones = jnp.full((LANES,), True)
        start(rank * CHUNK, 0)
        npairs = (n_chunks - rank + 2 * stride - 1) // (2 * stride)

        @pl.loop(0, npairs, init_carry=jnp.zeros((LANES,), jnp.float32))
        def pair_loop(j, acc):
            c0 = rank + j * 2 * stride
            c1 = c0 + stride
            v1 = c1 < n_chunks
            m1 = jnp.broadcast_to(v1, (LANES,))

            @pl.when(v1)
            def _():
                start(c1 * CHUNK, 1)

            wait(c0 * CHUNK, 0)
            acc = compute(0, acc, cols, ones)
            c2 = c0 + 2 * stride

            @pl.when(c2 < n_chunks)
            def _():
                start(c2 * CHUNK, 0)

            @pl.when(v1)
            def _():
                wait(c1 * CHUNK, 1)

            acc2 = compute(1, acc, cols, m1)
            return jnp.where(m1, acc2, acc)

        acc = pair_loop
        if tail:
            @pl.when(rank == 0)
            def _():
                base = n_chunks * CHUNK
                start(base, 0, tail)
                wait(base, 0, tail)
                accbuf[...] = compute(0, acc, cols, ones, n_edges=tail)

            @pl.when(rank != 0)
            def _():
                accbuf[...] = acc
        else:
            accbuf[...] = acc

    @pl.when(wid < PAIR_TILES)
    def _():
        pltpu.sync_copy(xcols.at[pl.ds(0 * np_rows, np_rows)], col0)
        pltpu.sync_copy(xcols.at[pl.ds(1 * np_rows, np_rows)], col1)
        run_class(wid, PAIR_TILES, (col0, col1))

    @pl.when((wid >= PAIR_TILES) & (wid < 2 * PAIR_TILES))
    def _():
        pltpu.sync_copy(xcols.at[pl.ds(2 * np_rows, np_rows)], col0)
        pltpu.sync_copy(xcols.at[pl.ds(3 * np_rows, np_rows)], col1)
        run_class(wid - PAIR_TILES, PAIR_TILES, (col0, col1))

    @pl.when(wid >= 2 * PAIR_TILES)
    def _():
        pltpu.sync_copy(xcols.at[pl.ds(4 * np_rows, np_rows)], col0)
        run_class(wid - 2 * PAIR_TILES, SINGLE_TILES, (col0,))

    pltpu.sync_copy(accbuf, out.at[wid])


def _finish_body(p_ref, w_ref, o_ref):
    o_ref[0] = jnp.sum(p_ref[...]) * w_ref[0]


def kernel(x, edge_index, W1, b1, W2, b2, w):
    n = x.shape[0]
    e = edge_index.shape[1]
    np_rows = ((n + LANES - 1) // LANES) * LANES
    n_chunks = e // CHUNK
    tail = e - n_chunks * CHUNK
    assert tail % LANES == 0, "tail remainder lanes not implemented"

    xcols = jnp.pad(x.astype(jnp.float32).T,
                    ((0, 0), (0, np_rows - n))).reshape(-1)
    ei = edge_index.astype(jnp.int32).reshape(-1)

    sc_call = pl.kernel(
        functools.partial(_sc_edge_dot_body, n_chunks=n_chunks, tail=tail,
                          e=e, np_rows=np_rows),
        out_type=jax.ShapeDtypeStruct((NUM_TILES, LANES), jnp.float32),
        mesh=plsc.VectorSubcoreMesh(core_axis_name="c", subcore_axis_name="s"),
        compiler_params=pltpu.CompilerParams(needs_layout_passes=False),
        scratch_types=[
            pltpu.VMEM((np_rows,), jnp.float32),
            pltpu.VMEM((np_rows,), jnp.float32),
            pltpu.VMEM((CHUNK,), jnp.int32),
            pltpu.VMEM((CHUNK,), jnp.int32),
            pltpu.VMEM((CHUNK,), jnp.int32),
            pltpu.VMEM((CHUNK,), jnp.int32),
            pltpu.VMEM((LANES,), jnp.float32),
            pltpu.SemaphoreType.DMA((2,)),
            pltpu.SemaphoreType.DMA((2,)),
        ],
    )
    partials = sc_call(xcols, ei)

    finish = pl.pallas_call(
        _finish_body,
        out_shape=jax.ShapeDtypeStruct((1,), jnp.float32),
        in_specs=[
            pl.BlockSpec(memory_space=pltpu.VMEM),
            pl.BlockSpec(memory_space=pltpu.SMEM),
        ],
        out_specs=pl.BlockSpec(memory_space=pltpu.SMEM),
    )
    return finish(partials, w.astype(jnp.float32))


# speedup vs baseline: 220.0771x; 1.1850x over previous
import functools

import jax
import jax.numpy as jnp
from jax import lax
from jax.experimental import pallas as pl
from jax.experimental.pallas import tpu as pltpu
from jax.experimental.pallas import tpu_sc as plsc

NUM_CORES = 2
NUM_SUBCORES = 16
NUM_TILES = NUM_CORES * NUM_SUBCORES
LANES = 16

NUM_DIMS = 5
PAIR_TILES = 12
SINGLE_TILES = 8
CHUNK = 7680
UNROLL = 8


def _sc_edge_dot_body(xcols, ei, out, col0, col1, sbuf0, sbuf1, dbuf0,
                      dbuf1, accbuf, ssem, dsem, *, n_chunks, tail, e, np_rows):
    sbufs = (sbuf0, sbuf1)
    dbufs = (dbuf0, dbuf1)
    wid = lax.axis_index("s") * NUM_CORES + lax.axis_index("c")
    accbuf[...] = jnp.zeros((LANES,), jnp.float32)

    def edge_copies(base, slot, size):
        return (
            pltpu.make_async_copy(ei.at[pl.ds(0, 1), pl.ds(base, size)],
                                  sbufs[slot].at[:, pl.ds(0, size)],
                                  ssem.at[slot]),
            pltpu.make_async_copy(ei.at[pl.ds(1, 1), pl.ds(base, size)],
                                  dbufs[slot].at[:, pl.ds(0, size)],
                                  dsem.at[slot]),
        )

    def start(base, slot, size=CHUNK):
        for cp in edge_copies(base, slot, size):
            cp.start()

    def wait(base, slot, size=CHUNK):
        for cp in edge_copies(base, slot, size):
            cp.wait()

    def compute(slot, acc, cols, mask, n_edges=CHUNK):
        sb = sbufs[slot]
        db = dbufs[slot]

        @plsc.parallel_loop(0, n_edges, step=LANES, unroll=UNROLL, carry=acc)
        def vec_loop(off, a):
            sv = sb[0, pl.ds(off, LANES)]
            tv = db[0, pl.ds(off, LANES)]
            for col in cols:
                a = a + (plsc.load_gather(col, [sv], mask=mask)
                         * plsc.load_gather(col, [tv], mask=mask))
            return a

        return vec_loop

    def run_class(rank, stride, cols):
        ones = jnp.full((LANES,), True)
        start(rank * CHUNK, 0)
        npairs = (n_chunks - rank + 2 * stride - 1) // (2 * stride)

        @pl.loop(0, npairs, init_carry=jnp.zeros((LANES,), jnp.float32))
        def pair_loop(j, acc):
            c0 = rank + j * 2 * stride
            c1 = c0 + stride
            v1 = c1 < n_chunks
            m1 = jnp.broadcast_to(v1, (LANES,))

            @pl.when(v1)
            def _():
                start(c1 * CHUNK, 1)

            wait(c0 * CHUNK, 0)
            acc = compute(0, acc, cols, ones)
            c2 = c0 + 2 * stride

            @pl.when(c2 < n_chunks)
            def _():
                start(c2 * CHUNK, 0)

            @pl.when(v1)
            def _():
                wait(c1 * CHUNK, 1)

            acc2 = compute(1, acc, cols, m1)
            return jnp.where(m1, acc2, acc)

        acc = pair_loop
        if tail:
            @pl.when(rank == 0)
            def _():
                base = n_chunks * CHUNK
                start(base, 0, tail)
                wait(base, 0, tail)
                accbuf[...] = compute(0, acc, cols, ones, n_edges=tail)

            @pl.when(rank != 0)
            def _():
                accbuf[...] = acc
        else:
            accbuf[...] = acc

    @pl.when(wid < PAIR_TILES)
    def _():
        pltpu.sync_copy(xcols.at[pl.ds(0 * np_rows, np_rows)], col0)
        pltpu.sync_copy(xcols.at[pl.ds(1 * np_rows, np_rows)], col1)
        run_class(wid, PAIR_TILES, (col0, col1))

    @pl.when((wid >= PAIR_TILES) & (wid < 2 * PAIR_TILES))
    def _():
        pltpu.sync_copy(xcols.at[pl.ds(2 * np_rows, np_rows)], col0)
        pltpu.sync_copy(xcols.at[pl.ds(3 * np_rows, np_rows)], col1)
        run_class(wid - PAIR_TILES, PAIR_TILES, (col0, col1))

    @pl.when(wid >= 2 * PAIR_TILES)
    def _():
        pltpu.sync_copy(xcols.at[pl.ds(4 * np_rows, np_rows)], col0)
        run_class(wid - 2 * PAIR_TILES, SINGLE_TILES, (col0,))

    pltpu.sync_copy(accbuf, out.at[wid])


def _finish_body(p_ref, w_ref, o_ref):
    o_ref[0] = jnp.sum(p_ref[...]) * w_ref[0]


def kernel(x, edge_index, W1, b1, W2, b2, w):
    n = x.shape[0]
    e = edge_index.shape[1]
    np_rows = ((n + LANES - 1) // LANES) * LANES
    n_chunks = e // CHUNK
    tail = e - n_chunks * CHUNK
    assert tail % LANES == 0, "tail remainder lanes not implemented"

    xcols = jnp.pad(x.astype(jnp.float32).T,
                    ((0, 0), (0, np_rows - n))).reshape(-1)
    ei = edge_index.astype(jnp.int32)

    sc_call = pl.kernel(
        functools.partial(_sc_edge_dot_body, n_chunks=n_chunks, tail=tail,
                          e=e, np_rows=np_rows),
        out_type=jax.ShapeDtypeStruct((NUM_TILES, LANES), jnp.float32),
        mesh=plsc.VectorSubcoreMesh(core_axis_name="c", subcore_axis_name="s"),
        compiler_params=pltpu.CompilerParams(needs_layout_passes=False),
        scratch_types=[
            pltpu.VMEM((np_rows,), jnp.float32),
            pltpu.VMEM((np_rows,), jnp.float32),
            pltpu.VMEM((1, CHUNK), jnp.int32),
            pltpu.VMEM((1, CHUNK), jnp.int32),
            pltpu.VMEM((1, CHUNK), jnp.int32),
            pltpu.VMEM((1, CHUNK), jnp.int32),
            pltpu.VMEM((LANES,), jnp.float32),
            pltpu.SemaphoreType.DMA((2,)),
            pltpu.SemaphoreType.DMA((2,)),
        ],
    )
    partials = sc_call(xcols, ei)

    finish = pl.pallas_call(
        _finish_body,
        out_shape=jax.ShapeDtypeStruct((1,), jnp.float32),
        in_specs=[
            pl.BlockSpec(memory_space=pltpu.VMEM),
            pl.BlockSpec(memory_space=pltpu.SMEM),
        ],
        out_specs=pl.BlockSpec(memory_space=pltpu.SMEM),
    )
    return finish(partials, w.astype(jnp.float32))
